# baseline (speedup 1.0000x reference)
import functools

import jax
import jax.numpy as jnp
from jax import lax
from jax.experimental import pallas as pl
from jax.experimental.pallas import tpu as pltpu
from jax.experimental.pallas import tpu_sc as plsc

TYPE_NUM = 1000000
TYPE_DIM = 64
BATCH = 16384
_SPLIT = 499968
_HALF = TYPE_NUM // 2

_INFO = plsc.get_sparse_core_info()
_NC = _INFO.num_cores
_NS = _INFO.num_subcores
_NW = _NC * _NS
_TOTAL = BATCH * 2
_IPW = _TOTAL // _NW
_CHUNK = 128
_NCHUNK = _IPW // _CHUNK

_BLK = 8064
_GRID = _SPLIT // _BLK


def _repack_kernel(src_lo, src_hi, dst):
    x = jnp.concatenate([src_lo[...], src_hi[...]], axis=0)
    dst[...] = jnp.transpose(x)


_repack = pl.pallas_call(
    _repack_kernel,
    grid=(_GRID,),
    in_specs=[
        pl.BlockSpec((TYPE_DIM, _BLK), lambda j: (0, j)),
        pl.BlockSpec((TYPE_DIM, _BLK), lambda j: (0, j + _GRID)),
    ],
    out_specs=pl.BlockSpec((_BLK, 2 * TYPE_DIM), lambda j: (j, 0)),
    out_shape=jax.ShapeDtypeStruct((_HALF, 2 * TYPE_DIM), jnp.float32),
)


def _tail_kernel(packed_ref, tsrc_ref, out_ref):
    tt = jnp.transpose(tsrc_ref[...])
    out_ref[:, :TYPE_DIM] = tt[:32, :]
    out_ref[:, TYPE_DIM:] = tt[32:, :]


_tail = pl.pallas_call(
    _tail_kernel,
    grid=(1,),
    in_specs=[
        pl.BlockSpec(memory_space=pl.ANY),
        pl.BlockSpec((TYPE_DIM, TYPE_DIM), lambda j: (0, 0)),
    ],
    out_specs=pl.BlockSpec((32, 2 * TYPE_DIM), lambda j: (_SPLIT // 32, 0)),
    out_shape=jax.ShapeDtypeStruct((_HALF, 2 * TYPE_DIM), jnp.float32),
    input_output_aliases={0: 0},
)


def _make_gather():
    mesh = plsc.VectorSubcoreMesh(core_axis_name="c", subcore_axis_name="s")

    @functools.partial(
        pl.kernel,
        mesh=mesh,
        compiler_params=pltpu.CompilerParams(
            use_tc_tiling_on_sc=False, needs_layout_passes=False),
        out_type=jax.ShapeDtypeStruct((_TOTAL, TYPE_DIM), jnp.float32),
        scratch_types=[
            pltpu.VMEM((_NCHUNK // 2, 2, _CHUNK), jnp.int32),
            pltpu.VMEM((_NCHUNK, _CHUNK), jnp.int32),
            pltpu.VMEM((2, _CHUNK, TYPE_DIM), jnp.float32),
            pltpu.SemaphoreType.DMA,
            pltpu.SemaphoreType.DMA,
        ],
    )
    def gather_kernel(table_hbm, vidx_hbm, out_hbm, vidx_v, hid_v, rows_v,
                      sem0, sem1):
        wid = lax.axis_index("s") * _NC + lax.axis_index("c")
        obase = wid * _IPW
        pltpu.sync_copy(vidx_hbm.at[pl.ds(wid * (_NCHUNK // 2), _NCHUNK // 2)],
                        vidx_v)
        for tt in range(_NCHUNK // 2):
            for j in range(2):
                for m in range(_CHUNK // 16):
                    v = vidx_v[tt, j, pl.ds(m * 16, 16)]
                    sel1 = jnp.where(
                        v < _SPLIT, jnp.int32(0),
                        jnp.where(v < 2 * _SPLIT + 32, jnp.int32(_SPLIT),
                                  jnp.int32(_SPLIT + 32)))
                    par = jnp.where(
                        v < _SPLIT, jnp.int32(0),
                        jnp.where(v < 2 * _SPLIT, jnp.int32(1),
                                  jnp.where(v < 2 * _SPLIT + 32, jnp.int32(0),
                                            jnp.int32(1))))
                    hid16 = 2 * (v - sel1) + par
                    ch = 2 * tt + (1 if m >= 4 else 0)
                    colbase = 2 * ((m * 16) % 64) + j
                    cols = colbase + 2 * lax.iota(jnp.int32, 16)
                    plsc.store_scatter(
                        hid_v, [jnp.full((16,), ch, jnp.int32), cols], hid16)
        sems = [sem0, sem1]
        copies = [None, None]
        copies[0] = pltpu.async_copy(
            table_hbm.at[hid_v.at[0]], rows_v.at[0], sems[0])
        for ch in range(_NCHUNK):
            nxt = (ch + 1) % 2
            if ch + 1 < _NCHUNK:
                copies[nxt] = pltpu.async_copy(
                    table_hbm.at[hid_v.at[ch + 1]], rows_v.at[nxt], sems[nxt])
            copies[ch % 2].wait()
            pltpu.sync_copy(
                rows_v.at[ch % 2],
                out_hbm.at[pl.ds(obase + ch * _CHUNK, _CHUNK)])

    return gather_kernel


_GATHER = _make_gather()


def kernel(inputs, type_matrix):
    table_t = jnp.transpose(type_matrix)
    packed = _repack(table_t, table_t)
    tail_src = lax.slice(table_t, (0, 2 * _SPLIT), (TYPE_DIM, TYPE_NUM))
    packed = _tail(packed, tail_src)
    flat = jnp.reshape(packed, (TYPE_NUM, TYPE_DIM))

    vidx = jnp.transpose(
        jnp.reshape(inputs.astype(jnp.int32), (_TOTAL // 256, _CHUNK, 2)),
        (0, 2, 1))

    out = _GATHER(flat, vidx)
    return jnp.reshape(out, (BATCH, 2 * TYPE_DIM))


# speedup vs baseline: 1.0274x; 1.0274x over previous
import functools

import jax
import jax.numpy as jnp
from jax import lax
from jax.experimental import pallas as pl
from jax.experimental.pallas import tpu as pltpu
from jax.experimental.pallas import tpu_sc as plsc

TYPE_NUM = 1000000
TYPE_DIM = 64
BATCH = 16384
_SPLIT = 499968
_HALF = TYPE_NUM // 2

_INFO = plsc.get_sparse_core_info()
_NC = _INFO.num_cores
_NS = _INFO.num_subcores
_NW = _NC * _NS
_TOTAL = BATCH * 2
_IPW = _TOTAL // _NW
_CHUNK = 128
_NCHUNK = _IPW // _CHUNK

_BLK = 23808
_GRID = _SPLIT // _BLK


def _repack_kernel(src_lo, src_hi, dst):
    x = jnp.concatenate([src_lo[...], src_hi[...]], axis=0)
    dst[...] = jnp.transpose(x)


_repack = pl.pallas_call(
    _repack_kernel,
    grid=(_GRID,),
    in_specs=[
        pl.BlockSpec((TYPE_DIM, _BLK), lambda j: (0, j)),
        pl.BlockSpec((TYPE_DIM, _BLK), lambda j: (0, j + _GRID)),
    ],
    out_specs=pl.BlockSpec((_BLK, 2 * TYPE_DIM), lambda j: (j, 0)),
    out_shape=jax.ShapeDtypeStruct((_HALF, 2 * TYPE_DIM), jnp.float32),
)


def _tail_kernel(packed_ref, tsrc_ref, out_ref):
    tt = jnp.transpose(tsrc_ref[...])
    out_ref[:, :TYPE_DIM] = tt[:32, :]
    out_ref[:, TYPE_DIM:] = tt[32:, :]


_tail = pl.pallas_call(
    _tail_kernel,
    grid=(1,),
    in_specs=[
        pl.BlockSpec(memory_space=pl.ANY),
        pl.BlockSpec((TYPE_DIM, TYPE_DIM), lambda j: (0, 0)),
    ],
    out_specs=pl.BlockSpec((32, 2 * TYPE_DIM), lambda j: (_SPLIT // 32, 0)),
    out_shape=jax.ShapeDtypeStruct((_HALF, 2 * TYPE_DIM), jnp.float32),
    input_output_aliases={0: 0},
)


def _make_gather():
    mesh = plsc.VectorSubcoreMesh(core_axis_name="c", subcore_axis_name="s")

    @functools.partial(
        pl.kernel,
        mesh=mesh,
        compiler_params=pltpu.CompilerParams(
            use_tc_tiling_on_sc=False, needs_layout_passes=False),
        out_type=jax.ShapeDtypeStruct((_TOTAL, TYPE_DIM), jnp.float32),
        scratch_types=[
            pltpu.VMEM((_NCHUNK // 2, 2, _CHUNK), jnp.int32),
            pltpu.VMEM((_NCHUNK, _CHUNK), jnp.int32),
            pltpu.VMEM((2, _CHUNK, TYPE_DIM), jnp.float32),
            pltpu.SemaphoreType.DMA,
            pltpu.SemaphoreType.DMA,
        ],
    )
    def gather_kernel(table_hbm, vidx_hbm, out_hbm, vidx_v, hid_v, rows_v,
                      sem0, sem1):
        wid = lax.axis_index("s") * _NC + lax.axis_index("c")
        obase = wid * _IPW
        pltpu.sync_copy(vidx_hbm.at[pl.ds(wid * (_NCHUNK // 2), _NCHUNK // 2)],
                        vidx_v)
        for tt in range(_NCHUNK // 2):
            for j in range(2):
                for m in range(_CHUNK // 16):
                    v = vidx_v[tt, j, pl.ds(m * 16, 16)]
                    sel1 = jnp.where(
                        v < _SPLIT, jnp.int32(0),
                        jnp.where(v < 2 * _SPLIT + 32, jnp.int32(_SPLIT),
                                  jnp.int32(_SPLIT + 32)))
                    par = jnp.where(
                        v < _SPLIT, jnp.int32(0),
                        jnp.where(v < 2 * _SPLIT, jnp.int32(1),
                                  jnp.where(v < 2 * _SPLIT + 32, jnp.int32(0),
                                            jnp.int32(1))))
                    hid16 = 2 * (v - sel1) + par
                    ch = 2 * tt + (1 if m >= 4 else 0)
                    colbase = 2 * ((m * 16) % 64) + j
                    cols = colbase + 2 * lax.iota(jnp.int32, 16)
                    plsc.store_scatter(
                        hid_v, [jnp.full((16,), ch, jnp.int32), cols], hid16)
        sems = [sem0, sem1]
        copies = [None, None]
        copies[0] = pltpu.async_copy(
            table_hbm.at[hid_v.at[0]], rows_v.at[0], sems[0])
        for ch in range(_NCHUNK):
            nxt = (ch + 1) % 2
            if ch + 1 < _NCHUNK:
                copies[nxt] = pltpu.async_copy(
                    table_hbm.at[hid_v.at[ch + 1]], rows_v.at[nxt], sems[nxt])
            copies[ch % 2].wait()
            pltpu.sync_copy(
                rows_v.at[ch % 2],
                out_hbm.at[pl.ds(obase + ch * _CHUNK, _CHUNK)])

    return gather_kernel


_GATHER = _make_gather()


def kernel(inputs, type_matrix):
    table_t = jnp.transpose(type_matrix)
    packed = _repack(table_t, table_t)
    tail_src = lax.slice(table_t, (0, 2 * _SPLIT), (TYPE_DIM, TYPE_NUM))
    packed = _tail(packed, tail_src)
    flat = jnp.reshape(packed, (TYPE_NUM, TYPE_DIM))

    vidx = jnp.transpose(
        jnp.reshape(inputs.astype(jnp.int32), (_TOTAL // 256, _CHUNK, 2)),
        (0, 2, 1))

    out = _GATHER(flat, vidx)
    return jnp.reshape(out, (BATCH, 2 * TYPE_DIM))


# speedup vs baseline: 1.0313x; 1.0038x over previous
import functools

import jax
import jax.numpy as jnp
from jax import lax
from jax.experimental import pallas as pl
from jax.experimental.pallas import tpu as pltpu
from jax.experimental.pallas import tpu_sc as plsc

TYPE_NUM = 1000000
TYPE_DIM = 64
BATCH = 16384
_SPLIT = 499968
_HALF = TYPE_NUM // 2

_INFO = plsc.get_sparse_core_info()
_NC = _INFO.num_cores
_NS = _INFO.num_subcores
_NW = _NC * _NS
_TOTAL = BATCH * 2
_IPW = _TOTAL // _NW
_CHUNK = 128
_NCHUNK = _IPW // _CHUNK

_BLK = 23808
_GRID = _SPLIT // _BLK


def _repack_kernel(src_lo, src_hi, dst):
    x = jnp.concatenate([src_lo[...], src_hi[...]], axis=0)
    dst[...] = jnp.transpose(x)


_repack = pl.pallas_call(
    _repack_kernel,
    grid=(_GRID,),
    in_specs=[
        pl.BlockSpec((TYPE_DIM, _BLK), lambda j: (0, j)),
        pl.BlockSpec((TYPE_DIM, _BLK), lambda j: (0, j + _GRID)),
    ],
    out_specs=pl.BlockSpec((_BLK, 2 * TYPE_DIM), lambda j: (j, 0)),
    out_shape=jax.ShapeDtypeStruct((_HALF, 2 * TYPE_DIM), jnp.float32),
)


def _tail_kernel(packed_ref, tsrc_ref, out_ref):
    tt = jnp.transpose(tsrc_ref[...])
    out_ref[:, :TYPE_DIM] = tt[:32, :]
    out_ref[:, TYPE_DIM:] = tt[32:, :]


_tail = pl.pallas_call(
    _tail_kernel,
    grid=(1,),
    in_specs=[
        pl.BlockSpec(memory_space=pl.ANY),
        pl.BlockSpec((TYPE_DIM, TYPE_DIM), lambda j: (0, 0)),
    ],
    out_specs=pl.BlockSpec((32, 2 * TYPE_DIM), lambda j: (_SPLIT // 32, 0)),
    out_shape=jax.ShapeDtypeStruct((_HALF, 2 * TYPE_DIM), jnp.float32),
    input_output_aliases={0: 0},
)


def _make_gather():
    mesh = plsc.VectorSubcoreMesh(core_axis_name="c", subcore_axis_name="s")

    @functools.partial(
        pl.kernel,
        mesh=mesh,
        compiler_params=pltpu.CompilerParams(
            use_tc_tiling_on_sc=False, needs_layout_passes=False),
        out_type=jax.ShapeDtypeStruct((_TOTAL, TYPE_DIM), jnp.float32),
        scratch_types=[
            pltpu.VMEM((_NCHUNK // 2, 2, _CHUNK), jnp.int32),
            pltpu.VMEM((_NCHUNK, _CHUNK), jnp.int32),
            pltpu.VMEM((2, _CHUNK, TYPE_DIM), jnp.float32),
            pltpu.SemaphoreType.DMA,
            pltpu.SemaphoreType.DMA,
            pltpu.SemaphoreType.DMA,
            pltpu.SemaphoreType.DMA,
        ],
    )
    def gather_kernel(table_hbm, vidx_hbm, out_hbm, vidx_v, hid_v, rows_v,
                      sem0, sem1, wsem0, wsem1):
        wid = lax.axis_index("s") * _NC + lax.axis_index("c")
        obase = wid * _IPW
        pltpu.sync_copy(vidx_hbm.at[pl.ds(wid * (_NCHUNK // 2), _NCHUNK // 2)],
                        vidx_v)
        for tt in range(_NCHUNK // 2):
            for j in range(2):
                for m in range(_CHUNK // 16):
                    v = vidx_v[tt, j, pl.ds(m * 16, 16)]
                    sel1 = jnp.where(
                        v < _SPLIT, jnp.int32(0),
                        jnp.where(v < 2 * _SPLIT + 32, jnp.int32(_SPLIT),
                                  jnp.int32(_SPLIT + 32)))
                    par = jnp.where(
                        v < _SPLIT, jnp.int32(0),
                        jnp.where(v < 2 * _SPLIT, jnp.int32(1),
                                  jnp.where(v < 2 * _SPLIT + 32, jnp.int32(0),
                                            jnp.int32(1))))
                    hid16 = 2 * (v - sel1) + par
                    ch = 2 * tt + (1 if m >= 4 else 0)
                    colbase = 2 * ((m * 16) % 64) + j
                    cols = colbase + 2 * lax.iota(jnp.int32, 16)
                    plsc.store_scatter(
                        hid_v, [jnp.full((16,), ch, jnp.int32), cols], hid16)
        sems = [sem0, sem1]
        wsems = [wsem0, wsem1]
        copies = [None, None]
        wcopies = [None, None]
        copies[0] = pltpu.async_copy(
            table_hbm.at[hid_v.at[0]], rows_v.at[0], sems[0])
        for ch in range(_NCHUNK):
            nxt = (ch + 1) % 2
            if ch + 1 < _NCHUNK:
                if wcopies[nxt] is not None:
                    wcopies[nxt].wait()
                copies[nxt] = pltpu.async_copy(
                    table_hbm.at[hid_v.at[ch + 1]], rows_v.at[nxt], sems[nxt])
            copies[ch % 2].wait()
            wcopies[ch % 2] = pltpu.async_copy(
                rows_v.at[ch % 2],
                out_hbm.at[pl.ds(obase + ch * _CHUNK, _CHUNK)],
                wsems[ch % 2])
        wcopies[0].wait()
        wcopies[1].wait()

    return gather_kernel


_GATHER = _make_gather()


def kernel(inputs, type_matrix):
    table_t = jnp.transpose(type_matrix)
    packed = _repack(table_t, table_t)
    tail_src = lax.slice(table_t, (0, 2 * _SPLIT), (TYPE_DIM, TYPE_NUM))
    packed = _tail(packed, tail_src)
    flat = jnp.reshape(packed, (TYPE_NUM, TYPE_DIM))

    vidx = jnp.transpose(
        jnp.reshape(inputs.astype(jnp.int32), (_TOTAL // 256, _CHUNK, 2)),
        (0, 2, 1))

    out = _GATHER(flat, vidx)
    return jnp.reshape(out, (BATCH, 2 * TYPE_DIM))


# speedup vs baseline: 1.0384x; 1.0069x over previous
import functools

import jax
import jax.numpy as jnp
from jax import lax
from jax.experimental import pallas as pl
from jax.experimental.pallas import tpu as pltpu
from jax.experimental.pallas import tpu_sc as plsc

TYPE_NUM = 1000000
TYPE_DIM = 64
BATCH = 16384
_SPLIT = 499968
_HALF = TYPE_NUM // 2

_INFO = plsc.get_sparse_core_info()
_NC = _INFO.num_cores
_NS = _INFO.num_subcores
_NW = _NC * _NS
_TOTAL = BATCH * 2
_IPW = _TOTAL // _NW
_CHUNK = 128
_NCHUNK = _IPW // _CHUNK

_BLK = 23808
_GRID = _SPLIT // _BLK


def _repack_kernel(src_lo, src_hi, dst):
    x = jnp.concatenate([src_lo[...], src_hi[...]], axis=0)
    dst[...] = jnp.transpose(x)


_repack = pl.pallas_call(
    _repack_kernel,
    grid=(_GRID,),
    in_specs=[
        pl.BlockSpec((TYPE_DIM, _BLK), lambda j: (0, j)),
        pl.BlockSpec((TYPE_DIM, _BLK), lambda j: (0, j + _GRID)),
    ],
    out_specs=pl.BlockSpec((_BLK, 2 * TYPE_DIM), lambda j: (j, 0)),
    out_shape=jax.ShapeDtypeStruct((_HALF, 2 * TYPE_DIM), jnp.float32),
)


def _tail_kernel(packed_ref, tsrc_ref, out_ref):
    tt = jnp.transpose(tsrc_ref[...])
    out_ref[:, :TYPE_DIM] = tt[:32, :]
    out_ref[:, TYPE_DIM:] = tt[32:, :]


_tail = pl.pallas_call(
    _tail_kernel,
    grid=(1,),
    in_specs=[
        pl.BlockSpec(memory_space=pl.ANY),
        pl.BlockSpec((TYPE_DIM, TYPE_DIM), lambda j: (0, 0)),
    ],
    out_specs=pl.BlockSpec((32, 2 * TYPE_DIM), lambda j: (_SPLIT // 32, 0)),
    out_shape=jax.ShapeDtypeStruct((_HALF, 2 * TYPE_DIM), jnp.float32),
    input_output_aliases={0: 0},
)


def _make_gather():
    mesh = plsc.VectorSubcoreMesh(core_axis_name="c", subcore_axis_name="s")

    @functools.partial(
        pl.kernel,
        mesh=mesh,
        compiler_params=pltpu.CompilerParams(
            use_tc_tiling_on_sc=False, needs_layout_passes=False),
        out_type=jax.ShapeDtypeStruct((_TOTAL, TYPE_DIM), jnp.float32),
        scratch_types=[
            pltpu.VMEM((_NCHUNK // 2, 2, _CHUNK), jnp.int32),
            pltpu.VMEM((_NCHUNK, _CHUNK), jnp.int32),
            pltpu.VMEM((_NCHUNK, _CHUNK, TYPE_DIM), jnp.float32),
            pltpu.SemaphoreType.DMA,
            pltpu.SemaphoreType.DMA,
        ],
    )
    def gather_kernel(table_hbm, vidx_hbm, out_hbm, vidx_v, hid_v, rows_v,
                      gsem, wsem):
        wid = lax.axis_index("s") * _NC + lax.axis_index("c")
        obase = wid * _IPW
        pltpu.sync_copy(vidx_hbm.at[pl.ds(wid * (_NCHUNK // 2), _NCHUNK // 2)],
                        vidx_v)
        for tt in range(_NCHUNK // 2):
            for j in range(2):
                for m in range(_CHUNK // 16):
                    v = vidx_v[tt, j, pl.ds(m * 16, 16)]
                    sel1 = jnp.where(
                        v < _SPLIT, jnp.int32(0),
                        jnp.where(v < 2 * _SPLIT + 32, jnp.int32(_SPLIT),
                                  jnp.int32(_SPLIT + 32)))
                    par = jnp.where(
                        v < _SPLIT, jnp.int32(0),
                        jnp.where(v < 2 * _SPLIT, jnp.int32(1),
                                  jnp.where(v < 2 * _SPLIT + 32, jnp.int32(0),
                                            jnp.int32(1))))
                    hid16 = 2 * (v - sel1) + par
                    ch = 2 * tt + (1 if m >= 4 else 0)
                    colbase = 2 * ((m * 16) % 64) + j
                    cols = colbase + 2 * lax.iota(jnp.int32, 16)
                    plsc.store_scatter(
                        hid_v, [jnp.full((16,), ch, jnp.int32), cols], hid16)
        copies = [
            pltpu.async_copy(table_hbm.at[hid_v.at[ch]], rows_v.at[ch], gsem)
            for ch in range(_NCHUNK)
        ]
        wcopies = []
        for ch in range(_NCHUNK):
            copies[ch].wait()
            wcopies.append(pltpu.async_copy(
                rows_v.at[ch],
                out_hbm.at[pl.ds(obase + ch * _CHUNK, _CHUNK)], wsem))
        for wc in wcopies:
            wc.wait()

    return gather_kernel


_GATHER = _make_gather()


def kernel(inputs, type_matrix):
    table_t = jnp.transpose(type_matrix)
    packed = _repack(table_t, table_t)
    tail_src = lax.slice(table_t, (0, 2 * _SPLIT), (TYPE_DIM, TYPE_NUM))
    packed = _tail(packed, tail_src)
    flat = jnp.reshape(packed, (TYPE_NUM, TYPE_DIM))

    vidx = jnp.transpose(
        jnp.reshape(inputs.astype(jnp.int32), (_TOTAL // 256, _CHUNK, 2)),
        (0, 2, 1))

    out = _GATHER(flat, vidx)
    return jnp.reshape(out, (BATCH, 2 * TYPE_DIM))
